# baseline (speedup 1.0000x reference)
import functools

import jax
import jax.numpy as jnp
from jax import lax
from jax.experimental import pallas as pl
from jax.experimental.pallas import tpu as pltpu
from jax.experimental.pallas import tpu_sc as plsc

_B = 4096
_L = 200
_D = 128
_C = 100

_NC, _NS = 2, 16
_NW = _NC * _NS
_RPW = _B // _NW
_G1, _G2 = 128, 72
_LANES = 16
_RBLK = 25


def _pooled_sc(x32, emb):
    _NBUF = 3
    mesh = plsc.VectorSubcoreMesh(core_axis_name="c", subcore_axis_name="s")

    @functools.partial(
        pl.kernel,
        out_type=jax.ShapeDtypeStruct((_B, _D), jnp.float32),
        mesh=mesh,
        scratch_types=[
            pltpu.VMEM((_RPW, _L), jnp.int32),
            pltpu.VMEM((_L, _D), jnp.float32),
            pltpu.VMEM((_L, _D), jnp.float32),
            pltpu.VMEM((_L, _D), jnp.float32),
            pltpu.VMEM((_RPW, _D), jnp.float32),
            pltpu.SemaphoreType.DMA,
            pltpu.SemaphoreType.DMA,
            pltpu.SemaphoreType.DMA,
        ],
    )
    def pool_kernel(x_hbm, emb_hbm, out_hbm, idx_v, buf0, buf1, buf2,
                    out_v, sem0, sem1, sem2):
        wid = lax.axis_index("s") * _NC + lax.axis_index("c")
        base = wid * _RPW
        pltpu.sync_copy(x_hbm.at[pl.ds(base, _RPW)], idx_v)

        bufs = (buf0, buf1, buf2)
        sems = (sem0, sem1, sem2)

        def issue(j, b):
            pltpu.async_copy(
                emb_hbm.at[idx_v.at[j, pl.ds(0, _G1)]],
                bufs[b].at[pl.ds(0, _G1)], sems[b])
            pltpu.async_copy(
                emb_hbm.at[idx_v.at[j, pl.ds(_G1, _G2)]],
                bufs[b].at[pl.ds(_G1, _G2)], sems[b])

        def wait(b):
            pltpu.make_async_copy(
                emb_hbm.at[pl.ds(0, _G1)], bufs[b].at[pl.ds(0, _G1)],
                sems[b]).wait()
            pltpu.make_async_copy(
                emb_hbm.at[pl.ds(0, _G2)], bufs[b].at[pl.ds(_G1, _G2)],
                sems[b]).wait()

        def accum(j, b):
            buf = bufs[b]
            zero = jnp.zeros((_LANES,), jnp.float32)
            for c in range(_D // _LANES):
                out_v[j, pl.ds(c * _LANES, _LANES)] = zero

            @pl.loop(0, _L, step=_RBLK)
            def _(r0):
                for c in range(_D // _LANES):
                    sl = pl.ds(c * _LANES, _LANES)
                    accs = [buf[r0 + t, sl] for t in range(4)]
                    for t in range(4, _RBLK):
                        accs[t % 4] = accs[t % 4] + buf[r0 + t, sl]
                    v = (accs[0] + accs[1]) + (accs[2] + accs[3])
                    out_v[j, sl] = out_v[j, sl] + v

        issue(0, 0)
        issue(1, 1)

        @pl.loop(0, _RPW - 2, step=_NBUF)
        def _(j):
            for b in range(_NBUF):
                jj = j + b
                issue(jj + 2, (b + 2) % _NBUF)
                wait(b)
                accum(jj, b)

        wait(0)
        accum(_RPW - 2, 0)
        wait(1)
        accum(_RPW - 1, 1)

        pltpu.sync_copy(out_v, out_hbm.at[pl.ds(base, _RPW)])

    return pool_kernel(x32, emb)


def _mlp_tc(pooled, w0t, b0, w1t, b1, w2t, b2, w3t, b3):
    blk = 512

    def body(h_ref, w0_ref, b0_ref, w1_ref, b1_ref, w2_ref, b2_ref,
             w3_ref, b3_ref, o_ref):
        h = h_ref[...]
        h = jnp.maximum(
            jnp.dot(h, w0_ref[...], preferred_element_type=jnp.float32)
            + b0_ref[...], 0.0)
        h = jnp.maximum(
            jnp.dot(h, w1_ref[...], preferred_element_type=jnp.float32)
            + b1_ref[...], 0.0)
        h = jnp.maximum(
            jnp.dot(h, w2_ref[...], preferred_element_type=jnp.float32)
            + b2_ref[...], 0.0)
        z = jnp.dot(h, w3_ref[...], preferred_element_type=jnp.float32) \
            + b3_ref[...]
        m = jnp.max(z, axis=1, keepdims=True)
        u = z - m
        lse = jnp.log(jnp.sum(jnp.exp(u), axis=1, keepdims=True))
        o_ref[...] = u - lse

    full = lambda a: pl.BlockSpec(a.shape, lambda i: (0,) * a.ndim)
    return pl.pallas_call(
        body,
        grid=(_B // blk,),
        in_specs=[
            pl.BlockSpec((blk, _D), lambda i: (i, 0)),
            full(w0t), full(b0), full(w1t), full(b1),
            full(w2t), full(b2), full(w3t), full(b3),
        ],
        out_specs=pl.BlockSpec((blk, _C), lambda i: (i, 0)),
        out_shape=jax.ShapeDtypeStruct((_B, _C), jnp.float32),
    )(pooled, w0t, b0, w1t, b1, w2t, b2, w3t, b3)


def kernel(x, emb, W0, b0, W1, b1, W2, b2, W3, b3):
    x32 = x.astype(jnp.int32)
    pooled = _pooled_sc(x32, emb)
    return _mlp_tc(
        pooled,
        W0.T * (1.0 / _L), b0.reshape(1, -1),
        W1.T, b1.reshape(1, -1),
        W2.T, b2.reshape(1, -1),
        W3.T, b3.reshape(1, -1),
    )


# speedup vs baseline: 1.0222x; 1.0222x over previous
import functools

import jax
import jax.numpy as jnp
from jax import lax
from jax.experimental import pallas as pl
from jax.experimental.pallas import tpu as pltpu
from jax.experimental.pallas import tpu_sc as plsc

_B = 4096
_L = 200
_D = 128
_C = 100

_NC, _NS = 2, 16
_NW = _NC * _NS
_RPW = _B // _NW
_G1, _G2 = 128, 72
_LANES = 16
_RBLK = 25


def _pooled_sc(x32, emb):
    _NBUF = 3
    mesh = plsc.VectorSubcoreMesh(core_axis_name="c", subcore_axis_name="s")

    @functools.partial(
        pl.kernel,
        out_type=jax.ShapeDtypeStruct((_B, _D), jnp.float32),
        mesh=mesh,
        scratch_types=[
            pltpu.VMEM((_RPW, _L), jnp.int32),
            pltpu.VMEM((_L, _D), jnp.float32),
            pltpu.VMEM((_L, _D), jnp.float32),
            pltpu.VMEM((_L, _D), jnp.float32),
            pltpu.VMEM((_RPW, _D), jnp.float32),
            pltpu.SemaphoreType.DMA,
            pltpu.SemaphoreType.DMA,
            pltpu.SemaphoreType.DMA,
        ],
    )
    def pool_kernel(x_hbm, emb_hbm, out_hbm, idx_v, buf0, buf1, buf2,
                    out_v, sem0, sem1, sem2):
        wid = lax.axis_index("s") * _NC + lax.axis_index("c")
        base = wid * _RPW
        pltpu.sync_copy(x_hbm.at[pl.ds(base, _RPW)], idx_v)

        bufs = (buf0, buf1, buf2)
        sems = (sem0, sem1, sem2)

        def issue(j, b):
            pltpu.async_copy(
                emb_hbm.at[idx_v.at[j, pl.ds(0, _G1)]],
                bufs[b].at[pl.ds(0, _G1)], sems[b])
            pltpu.async_copy(
                emb_hbm.at[idx_v.at[j, pl.ds(_G1, _G2)]],
                bufs[b].at[pl.ds(_G1, _G2)], sems[b])

        def wait(b):
            pltpu.make_async_copy(
                emb_hbm.at[pl.ds(0, _G1)], bufs[b].at[pl.ds(0, _G1)],
                sems[b]).wait()
            pltpu.make_async_copy(
                emb_hbm.at[pl.ds(0, _G2)], bufs[b].at[pl.ds(_G1, _G2)],
                sems[b]).wait()

        def accum(j, b):
            buf = bufs[b]
            zero = jnp.zeros((_LANES,), jnp.float32)
            for c in range(_D // _LANES):
                out_v[j, pl.ds(c * _LANES, _LANES)] = zero

            @pl.loop(0, _L, step=_RBLK)
            def _(r0):
                for c in range(_D // _LANES):
                    sl = pl.ds(c * _LANES, _LANES)
                    accs = [buf[r0 + t, sl] for t in range(4)]
                    for t in range(4, _RBLK):
                        accs[t % 4] = accs[t % 4] + buf[r0 + t, sl]
                    v = (accs[0] + accs[1]) + (accs[2] + accs[3])
                    out_v[j, sl] = out_v[j, sl] + v

        issue(0, 0)
        issue(1, 1)

        @pl.loop(0, _RPW - 2, step=_NBUF)
        def _(j):
            for b in range(_NBUF):
                jj = j + b
                issue(jj + 2, (b + 2) % _NBUF)
                wait(b)
                accum(jj, b)

        wait(0)
        accum(_RPW - 2, 0)
        wait(1)
        accum(_RPW - 1, 1)

        pltpu.sync_copy(out_v, out_hbm.at[pl.ds(base, _RPW)])

    return pool_kernel(x32, emb)


def _mlp_tc(pooled, w0t, b0, w1t, b1, w2t, b2, w3t, b3):
    blk = 2048

    def body(h_ref, w0_ref, b0_ref, w1_ref, b1_ref, w2_ref, b2_ref,
             w3_ref, b3_ref, o_ref):
        h = h_ref[...]
        h = jnp.maximum(
            jnp.dot(h, w0_ref[...], preferred_element_type=jnp.float32)
            + b0_ref[...], 0.0)
        h = jnp.maximum(
            jnp.dot(h, w1_ref[...], preferred_element_type=jnp.float32)
            + b1_ref[...], 0.0)
        h = jnp.maximum(
            jnp.dot(h, w2_ref[...], preferred_element_type=jnp.float32)
            + b2_ref[...], 0.0)
        z = jnp.dot(h, w3_ref[...], preferred_element_type=jnp.float32) \
            + b3_ref[...]
        m = jnp.max(z, axis=1, keepdims=True)
        u = z - m
        lse = jnp.log(jnp.sum(jnp.exp(u), axis=1, keepdims=True))
        o_ref[...] = u - lse

    full = lambda a: pl.BlockSpec(a.shape, lambda i: (0,) * a.ndim)
    return pl.pallas_call(
        body,
        grid=(_B // blk,),
        in_specs=[
            pl.BlockSpec((blk, _D), lambda i: (i, 0)),
            full(w0t), full(b0), full(w1t), full(b1),
            full(w2t), full(b2), full(w3t), full(b3),
        ],
        out_specs=pl.BlockSpec((blk, _C), lambda i: (i, 0)),
        out_shape=jax.ShapeDtypeStruct((_B, _C), jnp.float32),
    )(pooled, w0t, b0, w1t, b1, w2t, b2, w3t, b3)


def kernel(x, emb, W0, b0, W1, b1, W2, b2, W3, b3):
    x32 = x.astype(jnp.int32)
    pooled = _pooled_sc(x32, emb)
    return _mlp_tc(
        pooled,
        W0.T * (1.0 / _L), b0.reshape(1, -1),
        W1.T, b1.reshape(1, -1),
        W2.T, b2.reshape(1, -1),
        W3.T, b3.reshape(1, -1),
    )
